# Initial kernel scaffold; baseline (speedup 1.0000x reference)
import jax
import jax.numpy as jnp
from jax.experimental import pallas as pl


def kernel(features, W0, attn_l0, attn_r0, W1, attn_l1, attn_r1, edge_index):
    raise NotImplementedError("write your pallas kernel here")


# speedup vs baseline: 53.8728x; 53.8728x over previous
import functools

import jax
import jax.numpy as jnp
from jax import lax
from jax.experimental import pallas as pl
from jax.experimental.pallas import tpu as pltpu
from jax.experimental.pallas import tpu_sc as plsc

N_NODES = 10000
N_PAD = 10240
N_EDGES = 320000
NC = 2
NS = 16
NW = NC * NS
EW = N_EDGES // NW
B = 80
NCHUNK = EW // B
ZR = N_PAD // NS

f32 = jnp.float32


def _tc_prologue(xpad, W0, Mta, Mtb, K16):
    BLK = 1024

    def body(x_ref, w_ref, ma_ref, mb_ref, k_ref, h_ref, ta_ref, tb_ref,
             c_ref, mx_ref):
        i = pl.program_id(0)
        h = jnp.dot(x_ref[...], w_ref[...], preferred_element_type=f32)
        h_ref[...] = h
        ta = jnp.dot(h, ma_ref[...], preferred_element_type=f32)
        tb = jnp.dot(h, mb_ref[...], preferred_element_type=f32)
        ta_ref[...] = ta
        tb_ref[...] = tb
        cur = jnp.concatenate([jnp.max(ta, axis=0, keepdims=True),
                               jnp.max(tb, axis=0, keepdims=True)], axis=0)

        @pl.when(i == 0)
        def _():
            mx_ref[...] = jnp.full((2, 16), -1e30, f32)

        mx_ref[...] = jnp.maximum(mx_ref[...], cur)
        m = mx_ref[...]
        c_ref[...] = jnp.dot(m[0:1, :] + m[1:2, :], k_ref[...],
                             preferred_element_type=f32)

    return pl.pallas_call(
        body, grid=(N_PAD // BLK,),
        in_specs=[pl.BlockSpec((BLK, 128), lambda i: (i, 0)),
                  pl.BlockSpec((128, 128), lambda i: (0, 0)),
                  pl.BlockSpec((128, 16), lambda i: (0, 0)),
                  pl.BlockSpec((128, 16), lambda i: (0, 0)),
                  pl.BlockSpec((16, 16), lambda i: (0, 0))],
        out_specs=[pl.BlockSpec((BLK, 128), lambda i: (i, 0)),
                   pl.BlockSpec((BLK, 16), lambda i: (i, 0)),
                   pl.BlockSpec((BLK, 16), lambda i: (i, 0)),
                   pl.BlockSpec((1, 16), lambda i: (0, 0))],
        out_shape=[jax.ShapeDtypeStruct((N_PAD, 128), f32),
                   jax.ShapeDtypeStruct((N_PAD, 16), f32),
                   jax.ShapeDtypeStruct((N_PAD, 16), f32),
                   jax.ShapeDtypeStruct((1, 16), f32)],
        scratch_shapes=[pltpu.VMEM((2, 16), f32)],
    )(xpad, W0, Mta, Mtb, K16)


def _tc_mid(m0, m1, d0, d1, R8, W1p, Ma1, Mb1, K1):
    BLK = 1024

    def body(m0_ref, m1_ref, d0_ref, d1_ref, r8_ref, w1_ref, ma_ref, mb_ref,
             k1_ref, h1_ref, ta_ref, tb_ref, c_ref, mx_ref):
        i = pl.program_id(0)
        m = m0_ref[...] + m1_ref[...]
        d = d0_ref[...] + d1_ref[...]
        rep = jnp.dot(d, r8_ref[...], preferred_element_type=f32)
        rst = jnp.maximum(m / jnp.maximum(rep, 1e-30), 0.0)
        h1 = jnp.dot(rst, w1_ref[...], preferred_element_type=f32)
        h1_ref[...] = h1
        ta = jnp.dot(h1, ma_ref[...], preferred_element_type=f32)
        tb = jnp.dot(h1, mb_ref[...], preferred_element_type=f32)
        ta_ref[...] = ta
        tb_ref[...] = tb
        cur = jnp.concatenate([jnp.max(ta, axis=0, keepdims=True),
                               jnp.max(tb, axis=0, keepdims=True)], axis=0)

        @pl.when(i == 0)
        def _():
            mx_ref[...] = jnp.full((2, 16), -1e30, f32)

        mx_ref[...] = jnp.maximum(mx_ref[...], cur)
        m2 = mx_ref[...]
        c_ref[...] = jnp.dot(m2[0:1, :] + m2[1:2, :], k1_ref[...],
                             preferred_element_type=f32)

    return pl.pallas_call(
        body, grid=(N_PAD // BLK,),
        in_specs=[pl.BlockSpec((BLK, 128), lambda i: (i, 0)),
                  pl.BlockSpec((BLK, 128), lambda i: (i, 0)),
                  pl.BlockSpec((BLK, 16), lambda i: (i, 0)),
                  pl.BlockSpec((BLK, 16), lambda i: (i, 0)),
                  pl.BlockSpec((16, 128), lambda i: (0, 0)),
                  pl.BlockSpec((128, 48), lambda i: (0, 0)),
                  pl.BlockSpec((48, 16), lambda i: (0, 0)),
                  pl.BlockSpec((48, 16), lambda i: (0, 0)),
                  pl.BlockSpec((16, 16), lambda i: (0, 0))],
        out_specs=[pl.BlockSpec((BLK, 48), lambda i: (i, 0)),
                   pl.BlockSpec((BLK, 16), lambda i: (i, 0)),
                   pl.BlockSpec((BLK, 16), lambda i: (i, 0)),
                   pl.BlockSpec((1, 16), lambda i: (0, 0))],
        out_shape=[jax.ShapeDtypeStruct((N_PAD, 48), f32),
                   jax.ShapeDtypeStruct((N_PAD, 16), f32),
                   jax.ShapeDtypeStruct((N_PAD, 16), f32),
                   jax.ShapeDtypeStruct((1, 16), f32)],
        scratch_shapes=[pltpu.VMEM((2, 16), f32)],
    )(m0, m1, d0, d1, R8, W1p, Ma1, Mb1, K1)


def _tc_final(m0, m1, d0, d1, S48, R1):
    BLK = 1000

    def body(m0_ref, m1_ref, d0_ref, d1_ref, s_ref, r_ref, o_ref):
        m = m0_ref[...] + m1_ref[...]
        d = d0_ref[...] + d1_ref[...]
        num = jnp.dot(m, s_ref[...], preferred_element_type=f32)
        db = jnp.dot(d, r_ref[...], preferred_element_type=f32)
        o_ref[...] = num / jnp.maximum(db, 1e-30)

    return pl.pallas_call(
        body, grid=(N_NODES // BLK,),
        in_specs=[pl.BlockSpec((BLK, 48), lambda i: (i, 0)),
                  pl.BlockSpec((BLK, 48), lambda i: (i, 0)),
                  pl.BlockSpec((BLK, 16), lambda i: (i, 0)),
                  pl.BlockSpec((BLK, 16), lambda i: (i, 0)),
                  pl.BlockSpec((48, 40), lambda i: (0, 0)),
                  pl.BlockSpec((16, 40), lambda i: (0, 0))],
        out_specs=pl.BlockSpec((BLK, 40), lambda i: (i, 0)),
        out_shape=jax.ShapeDtypeStruct((N_NODES, 40), f32),
    )(m0, m1, d0, d1, S48, R1)


def _make_sc_edge_pass(ncols, per_head):
    njs = ncols // 16
    mesh = plsc.VectorSubcoreMesh(core_axis_name="c", subcore_axis_name="s",
                                  num_cores=NC, num_subcores=NS)
    out_type = (jax.ShapeDtypeStruct((NC, N_PAD, 16), f32),
                jax.ShapeDtypeStruct((NC, N_PAD, ncols), f32))
    scratch = [
        pltpu.VMEM((B,), jnp.int32),
        pltpu.VMEM((B,), jnp.int32),
        pltpu.VMEM((B, 16), f32),
        pltpu.VMEM((B, 16), f32),
        pltpu.VMEM((B, 16), f32),
        pltpu.VMEM((B, ncols), f32),
        pltpu.VMEM((16,), f32),
        pltpu.VMEM_SHARED((N_PAD, 16), f32),
        pltpu.VMEM_SHARED((N_PAD, ncols), f32),
        pltpu.SemaphoreType.DMA,
        pltpu.SemaphoreType.DMA,
    ]

    @functools.partial(pl.kernel, mesh=mesh, out_type=out_type,
                       scratch_types=scratch,
                       compiler_params=pltpu.CompilerParams(
                           use_tc_tiling_on_sc=False))
    def sc_pass(h_hbm, ta_hbm, tb_hbm, cvec_hbm, src_hbm, dst_hbm,
                den_hbm, msg_hbm,
                sidx, didx, r1, r2, eebuf, hbuf, cbuf, den_sp, msg_sp,
                semt, semh):
        cid = lax.axis_index("c")
        sid = lax.axis_index("s")
        w = sid * NC + cid
        zv = jnp.zeros((16,), f32)

        def zrow(rn, carry):
            for j in range(njs):
                hbuf[rn, pl.ds(j * 16, 16)] = zv
            eebuf[rn, :] = zv
            return carry

        lax.fori_loop(0, B, zrow, 0)
        zbase = sid * ZR
        for t in range(ZR // B):
            pltpu.sync_copy(eebuf, den_sp.at[pl.ds(zbase + t * B, B), :])
            pltpu.sync_copy(hbuf, msg_sp.at[pl.ds(zbase + t * B, B), :])
        plsc.subcore_barrier()
        pltpu.sync_copy(cvec_hbm.at[0], cbuf)
        cv = cbuf[...]

        def chunk(k, carry):
            base = w * EW + k * B
            pltpu.sync_copy(src_hbm.at[pl.ds(base, B)], sidx)
            pltpu.sync_copy(dst_hbm.at[pl.ds(base, B)], didx)
            cpa = pltpu.async_copy(ta_hbm.at[sidx], r1, semt)
            cpb = pltpu.async_copy(tb_hbm.at[didx], r2, semt)
            cph = pltpu.async_copy(h_hbm.at[sidx], hbuf, semh)
            cpa.wait()
            cpb.wait()

            def erow(rn, c2):
                v = r1[rn, :] + r2[rn, :]
                v = jnp.where(v > 0.0, v, 0.2 * v)
                eebuf[rn, :] = jnp.exp(v - cv)
                return c2

            lax.fori_loop(0, B, erow, 0)
            pltpu.sync_copy(eebuf, den_sp.at[didx], add=True)
            cph.wait()

            def mrow(rn, c3):
                ev = eebuf[rn, :]
                for j in range(njs):
                    s = ev[j if per_head else 0]
                    hbuf[rn, pl.ds(j * 16, 16)] = hbuf[rn, pl.ds(j * 16, 16)] * s
                return c3

            lax.fori_loop(0, B, mrow, 0)
            pltpu.sync_copy(hbuf, msg_sp.at[didx], add=True)
            return carry

        lax.fori_loop(0, NCHUNK, chunk, 0)
        plsc.subcore_barrier()
        pltpu.sync_copy(den_sp.at[pl.ds(zbase, ZR), :],
                        den_hbm.at[cid, pl.ds(zbase, ZR), :])
        pltpu.sync_copy(msg_sp.at[pl.ds(zbase, ZR), :],
                        msg_hbm.at[cid, pl.ds(zbase, ZR), :])

    return sc_pass


_sc_pass_l0 = _make_sc_edge_pass(128, True)
_sc_pass_l1 = _make_sc_edge_pass(48, False)


def kernel(features, W0, attn_l0, attn_r0, W1, attn_l1, attn_r1, edge_index):
    xpad = jnp.zeros((N_PAD, 128), f32).at[:N_NODES].set(features)
    eye8 = jnp.eye(8, dtype=f32)
    z8 = jnp.zeros((128, 8), f32)
    Mta = jnp.concatenate(
        [(attn_l0[:, :, None] * eye8[:, None, :]).reshape(128, 8), z8], axis=1)
    Mtb = jnp.concatenate(
        [(attn_r0[:, :, None] * eye8[:, None, :]).reshape(128, 8), z8], axis=1)
    K16 = jnp.tile(eye8, (2, 2))
    R8 = jnp.concatenate([jnp.kron(eye8, jnp.ones((1, 16), f32)),
                          jnp.zeros((8, 128), f32)], axis=0)
    W1p = jnp.concatenate([W1, jnp.zeros((128, 8), f32)], axis=1)
    Ma1 = jnp.zeros((48, 16), f32).at[:40, 0].set(attn_l1[0])
    Mb1 = jnp.zeros((48, 16), f32).at[:40, 0].set(attn_r1[0])
    K1 = jnp.zeros((16, 16), f32).at[0, :].set(1.0)
    S48 = jnp.eye(48, dtype=f32)[:, :40]
    R1 = jnp.zeros((16, 40), f32).at[0, :].set(1.0)

    src = edge_index[0]
    dst = edge_index[1]
    h, ta, tb, cvec = _tc_prologue(xpad, W0, Mta, Mtb, K16)
    den, msg = _sc_pass_l0(h, ta, tb, cvec, src, dst)
    h1p, ta1, tb1, c1vec = _tc_mid(msg[0], msg[1], den[0], den[1],
                                   R8, W1p, Ma1, Mb1, K1)
    den1, msg1 = _sc_pass_l1(h1p, ta1, tb1, c1vec, src, dst)
    out = _tc_final(msg1[0, :N_NODES], msg1[1, :N_NODES],
                    den1[0, :N_NODES], den1[1, :N_NODES], S48, R1)
    return out


# speedup vs baseline: 58.6881x; 1.0894x over previous
import functools

import jax
import jax.numpy as jnp
from jax import lax
from jax.experimental import pallas as pl
from jax.experimental.pallas import tpu as pltpu
from jax.experimental.pallas import tpu_sc as plsc

N_NODES = 10000
N_PAD = 10240
N_EDGES = 320000
NC = 2
NS = 16
NW = NC * NS
EW = N_EDGES // NW
B = 80
NCHUNK = EW // B
ZR = N_PAD // NS

f32 = jnp.float32


def _tc_prologue(xpad, W0, Mta, Mtb, K16):
    BLK = 1024

    def body(x_ref, w_ref, ma_ref, mb_ref, k_ref, h_ref, ta_ref, tb_ref,
             c_ref, mx_ref):
        i = pl.program_id(0)
        h = jnp.dot(x_ref[...], w_ref[...], preferred_element_type=f32)
        h_ref[...] = h
        ta = jnp.dot(h, ma_ref[...], preferred_element_type=f32)
        tb = jnp.dot(h, mb_ref[...], preferred_element_type=f32)
        ta_ref[...] = ta
        tb_ref[...] = tb
        cur = jnp.concatenate([jnp.max(ta, axis=0, keepdims=True),
                               jnp.max(tb, axis=0, keepdims=True)], axis=0)

        @pl.when(i == 0)
        def _():
            mx_ref[...] = jnp.full((2, 16), -1e30, f32)

        mx_ref[...] = jnp.maximum(mx_ref[...], cur)
        m = mx_ref[...]
        c_ref[...] = jnp.dot(m[0:1, :] + m[1:2, :], k_ref[...],
                             preferred_element_type=f32)

    return pl.pallas_call(
        body, grid=(N_PAD // BLK,),
        in_specs=[pl.BlockSpec((BLK, 128), lambda i: (i, 0)),
                  pl.BlockSpec((128, 128), lambda i: (0, 0)),
                  pl.BlockSpec((128, 16), lambda i: (0, 0)),
                  pl.BlockSpec((128, 16), lambda i: (0, 0)),
                  pl.BlockSpec((16, 16), lambda i: (0, 0))],
        out_specs=[pl.BlockSpec((BLK, 128), lambda i: (i, 0)),
                   pl.BlockSpec((BLK, 16), lambda i: (i, 0)),
                   pl.BlockSpec((BLK, 16), lambda i: (i, 0)),
                   pl.BlockSpec((1, 16), lambda i: (0, 0))],
        out_shape=[jax.ShapeDtypeStruct((N_PAD, 128), f32),
                   jax.ShapeDtypeStruct((N_PAD, 16), f32),
                   jax.ShapeDtypeStruct((N_PAD, 16), f32),
                   jax.ShapeDtypeStruct((1, 16), f32)],
        scratch_shapes=[pltpu.VMEM((2, 16), f32)],
    )(xpad, W0, Mta, Mtb, K16)


def _tc_mid(m0, m1, d0, d1, R8, W1p, Ma1, Mb1, K1):
    BLK = 1024

    def body(m0_ref, m1_ref, d0_ref, d1_ref, r8_ref, w1_ref, ma_ref, mb_ref,
             k1_ref, h1_ref, ta_ref, tb_ref, c_ref, mx_ref):
        i = pl.program_id(0)
        m = m0_ref[...] + m1_ref[...]
        d = d0_ref[...] + d1_ref[...]
        rep = jnp.dot(d, r8_ref[...], preferred_element_type=f32)
        rst = jnp.maximum(m / jnp.maximum(rep, 1e-30), 0.0)
        h1 = jnp.dot(rst, w1_ref[...], preferred_element_type=f32)
        h1_ref[...] = h1
        ta = jnp.dot(h1, ma_ref[...], preferred_element_type=f32)
        tb = jnp.dot(h1, mb_ref[...], preferred_element_type=f32)
        ta_ref[...] = ta
        tb_ref[...] = tb
        cur = jnp.concatenate([jnp.max(ta, axis=0, keepdims=True),
                               jnp.max(tb, axis=0, keepdims=True)], axis=0)

        @pl.when(i == 0)
        def _():
            mx_ref[...] = jnp.full((2, 16), -1e30, f32)

        mx_ref[...] = jnp.maximum(mx_ref[...], cur)
        m2 = mx_ref[...]
        c_ref[...] = jnp.dot(m2[0:1, :] + m2[1:2, :], k1_ref[...],
                             preferred_element_type=f32)

    return pl.pallas_call(
        body, grid=(N_PAD // BLK,),
        in_specs=[pl.BlockSpec((BLK, 128), lambda i: (i, 0)),
                  pl.BlockSpec((BLK, 128), lambda i: (i, 0)),
                  pl.BlockSpec((BLK, 16), lambda i: (i, 0)),
                  pl.BlockSpec((BLK, 16), lambda i: (i, 0)),
                  pl.BlockSpec((16, 128), lambda i: (0, 0)),
                  pl.BlockSpec((128, 48), lambda i: (0, 0)),
                  pl.BlockSpec((48, 16), lambda i: (0, 0)),
                  pl.BlockSpec((48, 16), lambda i: (0, 0)),
                  pl.BlockSpec((16, 16), lambda i: (0, 0))],
        out_specs=[pl.BlockSpec((BLK, 48), lambda i: (i, 0)),
                   pl.BlockSpec((BLK, 16), lambda i: (i, 0)),
                   pl.BlockSpec((BLK, 16), lambda i: (i, 0)),
                   pl.BlockSpec((1, 16), lambda i: (0, 0))],
        out_shape=[jax.ShapeDtypeStruct((N_PAD, 48), f32),
                   jax.ShapeDtypeStruct((N_PAD, 16), f32),
                   jax.ShapeDtypeStruct((N_PAD, 16), f32),
                   jax.ShapeDtypeStruct((1, 16), f32)],
        scratch_shapes=[pltpu.VMEM((2, 16), f32)],
    )(m0, m1, d0, d1, R8, W1p, Ma1, Mb1, K1)


def _tc_final(m0, m1, dall, S48, R1):
    BLK = 1000

    def body(m0_ref, m1_ref, dall_ref, s_ref, r_ref, o_ref):
        m = m0_ref[...] + m1_ref[...]
        d = jnp.sum(dall_ref[...], axis=0)
        num = jnp.dot(m, s_ref[...], preferred_element_type=f32)
        db = jnp.dot(d, r_ref[...], preferred_element_type=f32)
        o_ref[...] = num / jnp.maximum(db, 1e-30)

    return pl.pallas_call(
        body, grid=(N_NODES // BLK,),
        in_specs=[pl.BlockSpec((BLK, 48), lambda i: (i, 0)),
                  pl.BlockSpec((BLK, 48), lambda i: (i, 0)),
                  pl.BlockSpec((NW, BLK, 8), lambda i: (0, i, 0)),
                  pl.BlockSpec((48, 40), lambda i: (0, 0)),
                  pl.BlockSpec((8, 40), lambda i: (0, 0))],
        out_specs=pl.BlockSpec((BLK, 40), lambda i: (i, 0)),
        out_shape=jax.ShapeDtypeStruct((N_NODES, 40), f32),
    )(m0, m1, dall, S48, R1)


def _make_sc_edge_pass(ncols, per_head, den_spmem):
    njs = ncols // 16
    mesh = plsc.VectorSubcoreMesh(core_axis_name="c", subcore_axis_name="s",
                                  num_cores=NC, num_subcores=NS)
    if den_spmem:
        den_out = jax.ShapeDtypeStruct((NC, N_PAD, 16), f32)
        den_scr = pltpu.VMEM_SHARED((N_PAD, 16), f32)
    else:
        den_out = jax.ShapeDtypeStruct((NC, NS, N_PAD * 8), f32)
        den_scr = pltpu.VMEM((N_PAD * 8,), f32)
    out_type = (den_out, jax.ShapeDtypeStruct((NC, N_PAD, ncols), f32))
    eebuf_scr = (pltpu.VMEM((2, B, 16), f32) if den_spmem
                 else pltpu.VMEM((1, 1, 16), f32))
    scratch = [
        pltpu.VMEM((4, B), jnp.int32),
        pltpu.VMEM((4, B), jnp.int32),
        pltpu.VMEM((2, B, 16), f32),
        pltpu.VMEM((2, B, 16), f32),
        eebuf_scr,
        pltpu.VMEM((2, B, ncols), f32),
        pltpu.VMEM((16,), f32),
        den_scr,
        pltpu.VMEM_SHARED((N_PAD, ncols), f32),
    ] + [pltpu.SemaphoreType.DMA] * 12

    @functools.partial(pl.kernel, mesh=mesh, out_type=out_type,
                       scratch_types=scratch,
                       compiler_params=pltpu.CompilerParams(
                           use_tc_tiling_on_sc=False,
                           needs_layout_passes=False))
    def sc_pass(h_hbm, ta_hbm, tb_hbm, cvec_hbm, src_hbm, dst_hbm,
                den_hbm, msg_hbm,
                sidx, didx, r1, r2, eebuf, hbuf, cbuf, den, msg_sp,
                semt0, semt1, semh0, semh1,
                semm0, semm1, semd0, semd1,
                semi0, semi1, semi2, semi3):
        semt = (semt0, semt1)
        semh = (semh0, semh1)
        semm = (semm0, semm1)
        semd = (semd0, semd1)
        semi = (semi0, semi1, semi2, semi3)
        cid = lax.axis_index("c")
        sid = lax.axis_index("s")
        w = sid * NC + cid
        zv = jnp.zeros((16,), f32)
        lanes = jnp.arange(16, dtype=jnp.int32)
        m8 = lanes < 8

        def zrow(rn, carry):
            for j in range(njs):
                hbuf[0, rn, pl.ds(j * 16, 16)] = zv
            if den_spmem:
                eebuf[0, rn, :] = zv
            return carry

        lax.fori_loop(0, B, zrow, 0)
        zbase = sid * ZR
        if den_spmem:
            for t in range(ZR // B):
                pltpu.sync_copy(eebuf.at[0],
                                den.at[pl.ds(zbase + t * B, B), :])
        else:
            def zden(i, carry):
                den[pl.ds(16 * i, 16)] = zv
                return carry

            lax.fori_loop(0, N_PAD * 8 // 16, zden, 0)
        for t in range(ZR // B):
            pltpu.sync_copy(hbuf.at[0], msg_sp.at[pl.ds(zbase + t * B, B), :])
        plsc.subcore_barrier()
        pltpu.sync_copy(cvec_hbm.at[0], cbuf)
        cv = cbuf[...]

        def fire_idx(k2, q):
            base = w * EW + k2 * B
            pltpu.async_copy(src_hbm.at[pl.ds(base, B)], sidx.at[q], semi[q])
            pltpu.async_copy(dst_hbm.at[pl.ds(base, B)], didx.at[q], semi[q])

        def wait_idx(q):
            pltpu.make_async_copy(src_hbm.at[pl.ds(0, B)], sidx.at[q],
                                  semi[q]).wait()
            pltpu.make_async_copy(dst_hbm.at[pl.ds(0, B)], didx.at[q],
                                  semi[q]).wait()

        def fire_gather(q, e, s):
            pltpu.async_copy(ta_hbm.at[sidx.at[q]], r1.at[e], semt[e])
            pltpu.async_copy(tb_hbm.at[didx.at[q]], r2.at[e], semt[e])
            pltpu.async_copy(h_hbm.at[sidx.at[q]], hbuf.at[s], semh[s])

        def wait_gather(e, s):
            pltpu.make_async_copy(ta_hbm.at[sidx.at[0]], r1.at[e],
                                  semt[e]).wait()
            pltpu.make_async_copy(tb_hbm.at[didx.at[0]], r2.at[e],
                                  semt[e]).wait()
            pltpu.make_async_copy(h_hbm.at[sidx.at[0]], hbuf.at[s],
                                  semh[s]).wait()

        def drain_msg(s):
            pltpu.make_async_copy(hbuf.at[s], msg_sp.at[didx.at[0]],
                                  semm[s]).wait()

        def drain_den(e):
            pltpu.make_async_copy(eebuf.at[e], den.at[didx.at[0]],
                                  semd[e]).wait()

        def edge(e, dv, j, rn):
            v = r1[e, rn, :] + r2[e, rn, :]
            v = jnp.where(v > 0.0, v, 0.2 * v)
            ee = jnp.exp(v - cv)
            if den_spmem:
                eebuf[e, rn, :] = ee
            else:
                r1[e, rn, :] = ee
                idxv = dv[j] * 8 + lanes
                plsc.addupdate_scatter(den, [idxv], ee, mask=m8)

        def process(k, km, dm, dd, fi, nx):
            s = km & 1
            s1 = (km + 1) & 1
            e = km & 1
            e1 = (km + 1) & 1
            qi = km & 3
            qi1 = (km + 1) & 3
            qi2 = (km + 2) & 3
            if dm:
                drain_msg(s1)
            if fi:
                fire_idx(k + 2, qi2)
            if nx:
                wait_idx(qi1)
                fire_gather(qi1, e1, s1)
            wait_gather(e, s)
            if dd and den_spmem:
                drain_den(e1)

            def egrp(g, c2):
                dv = didx[qi, pl.ds(g * 16, 16)]
                for j in range(16):
                    edge(e, dv, j, g * 16 + j)
                return c2

            lax.fori_loop(0, B // 16, egrp, 0)
            if B % 16:
                dvt = didx[qi, pl.ds(B - 16, 16)]
                for j in range(16 - (B % 16), 16):
                    edge(e, dvt, j, B - 16 + j)
            if den_spmem:
                pltpu.async_copy(eebuf.at[e], den.at[didx.at[qi]], semd[e],
                                 add=True)

            eref = eebuf if den_spmem else r1

            def mrow(rn, c3):
                ev = eref[e, rn, :]
                for j in range(njs):
                    sc = ev[j if per_head else 0]
                    hbuf[s, rn, pl.ds(j * 16, 16)] = (
                        hbuf[s, rn, pl.ds(j * 16, 16)] * sc)
                return c3

            lax.fori_loop(0, B, mrow, 0, unroll=2)
            pltpu.async_copy(hbuf.at[s], msg_sp.at[didx.at[qi]], semm[s],
                             add=True)

        fire_idx(0, 0)
        fire_idx(1, 1)
        wait_idx(0)
        fire_gather(0, 0, 0)
        process(0, 0, False, False, True, True)
        process(1, 1, True, True, True, True)
        process(2, 2, True, True, True, True)
        process(3, 3, True, True, True, True)
        NSTEADY = (NCHUNK - 4 - 3) // 4

        def body4(g, carry):
            kb = 4 + 4 * g
            for j in range(4):
                process(kb + j, 4 + j, True, True, True, True)
            return carry

        lax.fori_loop(0, NSTEADY, body4, 0)
        for kp in range(4 + 4 * NSTEADY, NCHUNK):
            process(kp, kp, True, True,
                    kp + 2 <= NCHUNK - 1, kp + 1 <= NCHUNK - 1)
        drain_msg((NCHUNK - 1) & 1)
        if den_spmem:
            drain_den((NCHUNK - 1) & 1)
            pltpu.sync_copy(den.at[pl.ds(zbase, ZR), :],
                            den_hbm.at[cid, pl.ds(zbase, ZR), :])
        else:
            pltpu.sync_copy(den, den_hbm.at[cid, sid])
        plsc.subcore_barrier()
        pltpu.sync_copy(msg_sp.at[pl.ds(zbase, ZR), :],
                        msg_hbm.at[cid, pl.ds(zbase, ZR), :])

    return sc_pass


_sc_pass_l0 = _make_sc_edge_pass(128, True, True)
_sc_pass_l1 = _make_sc_edge_pass(48, False, False)


def kernel(features, W0, attn_l0, attn_r0, W1, attn_l1, attn_r1, edge_index):
    xpad = jnp.zeros((N_PAD, 128), f32).at[:N_NODES].set(features)
    eye8 = jnp.eye(8, dtype=f32)
    z8 = jnp.zeros((128, 8), f32)
    Mta = jnp.concatenate(
        [(attn_l0[:, :, None] * eye8[:, None, :]).reshape(128, 8), z8], axis=1)
    Mtb = jnp.concatenate(
        [(attn_r0[:, :, None] * eye8[:, None, :]).reshape(128, 8), z8], axis=1)
    K16 = jnp.tile(eye8, (2, 2))
    R8 = jnp.concatenate([jnp.kron(eye8, jnp.ones((1, 16), f32)),
                          jnp.zeros((8, 128), f32)], axis=0)
    W1p = jnp.concatenate([W1, jnp.zeros((128, 8), f32)], axis=1)
    Ma1 = jnp.zeros((48, 16), f32).at[:40, 0].set(attn_l1[0])
    Mb1 = jnp.zeros((48, 16), f32).at[:40, 0].set(attn_r1[0])
    K1 = jnp.zeros((16, 16), f32).at[0, :].set(1.0)
    S48 = jnp.eye(48, dtype=f32)[:, :40]
    R1 = jnp.zeros((8, 40), f32).at[0, :].set(1.0)

    src = edge_index[0]
    dst = edge_index[1]
    h, ta, tb, cvec = _tc_prologue(xpad, W0, Mta, Mtb, K16)
    den, msg = _sc_pass_l0(h, ta, tb, cvec, src, dst)
    h1p, ta1, tb1, c1vec = _tc_mid(msg[0], msg[1], den[0], den[1],
                                   R8, W1p, Ma1, Mb1, K1)
    den1, msg1 = _sc_pass_l1(h1p, ta1, tb1, c1vec, src, dst)
    dall1 = den1.reshape(NW, N_PAD, 8)[:, :N_NODES]
    out = _tc_final(msg1[0, :N_NODES], msg1[1, :N_NODES], dall1, S48, R1)
    return out


# speedup vs baseline: 108.3874x; 1.8468x over previous
import functools

import jax
import jax.numpy as jnp
from jax import lax
from jax.experimental import pallas as pl
from jax.experimental.pallas import tpu as pltpu
from jax.experimental.pallas import tpu_sc as plsc

N_NODES = 10000
N_PAD = 10240
N_EDGES = 320000
NC = 2
NS = 16
NW = NC * NS
EW = N_EDGES // NW
B = 80
NCHUNK = EW // B
ZR = N_PAD // NS

f32 = jnp.float32


def _tc_prologue(xpad, W0, Mta, Mtb, K16):
    BLK = 1024

    def body(x_ref, w_ref, ma_ref, mb_ref, k_ref, h_ref, ta_ref, tb_ref,
             c_ref, mx_ref):
        i = pl.program_id(0)
        h = jnp.dot(x_ref[...], w_ref[...], preferred_element_type=f32)
        h_ref[...] = h
        ta = jnp.dot(h, ma_ref[...], preferred_element_type=f32)
        tb = jnp.dot(h, mb_ref[...], preferred_element_type=f32)
        ta_ref[...] = ta
        tb_ref[...] = tb
        cur = jnp.concatenate([jnp.max(ta, axis=0, keepdims=True),
                               jnp.max(tb, axis=0, keepdims=True)], axis=0)

        @pl.when(i == 0)
        def _():
            mx_ref[...] = jnp.full((2, 16), -1e30, f32)

        mx_ref[...] = jnp.maximum(mx_ref[...], cur)
        m = mx_ref[...]
        c_ref[...] = jnp.dot(m[0:1, :] + m[1:2, :], k_ref[...],
                             preferred_element_type=f32)

    return pl.pallas_call(
        body, grid=(N_PAD // BLK,),
        in_specs=[pl.BlockSpec((BLK, 128), lambda i: (i, 0)),
                  pl.BlockSpec((128, 128), lambda i: (0, 0)),
                  pl.BlockSpec((128, 16), lambda i: (0, 0)),
                  pl.BlockSpec((128, 16), lambda i: (0, 0)),
                  pl.BlockSpec((16, 16), lambda i: (0, 0))],
        out_specs=[pl.BlockSpec((BLK, 128), lambda i: (i, 0)),
                   pl.BlockSpec((BLK, 16), lambda i: (i, 0)),
                   pl.BlockSpec((BLK, 16), lambda i: (i, 0)),
                   pl.BlockSpec((1, 16), lambda i: (0, 0))],
        out_shape=[jax.ShapeDtypeStruct((N_PAD, 128), f32),
                   jax.ShapeDtypeStruct((N_PAD, 16), f32),
                   jax.ShapeDtypeStruct((N_PAD, 16), f32),
                   jax.ShapeDtypeStruct((1, 16), f32)],
        scratch_shapes=[pltpu.VMEM((2, 16), f32)],
    )(xpad, W0, Mta, Mtb, K16)


def _tc_mid(m0, m1, d0, d1, R8, W1p, Ma1, Mb1, K1):
    BLK = 1024

    def body(m0_ref, m1_ref, d0_ref, d1_ref, r8_ref, w1_ref, ma_ref, mb_ref,
             k1_ref, h1_ref, ta_ref, tb_ref, c_ref, mx_ref):
        i = pl.program_id(0)
        m = m0_ref[...] + m1_ref[...]
        d = d0_ref[...] + d1_ref[...]
        rep = jnp.dot(d, r8_ref[...], preferred_element_type=f32)
        rst = jnp.maximum(m / jnp.maximum(rep, 1e-30), 0.0)
        h1 = jnp.dot(rst, w1_ref[...], preferred_element_type=f32)
        h1_ref[...] = h1
        ta = jnp.dot(h1, ma_ref[...], preferred_element_type=f32)
        tb = jnp.dot(h1, mb_ref[...], preferred_element_type=f32)
        ta_ref[...] = ta
        tb_ref[...] = tb
        cur = jnp.concatenate([jnp.max(ta, axis=0, keepdims=True),
                               jnp.max(tb, axis=0, keepdims=True)], axis=0)

        @pl.when(i == 0)
        def _():
            mx_ref[...] = jnp.full((2, 16), -1e30, f32)

        mx_ref[...] = jnp.maximum(mx_ref[...], cur)
        m2 = mx_ref[...]
        c_ref[...] = jnp.dot(m2[0:1, :] + m2[1:2, :], k1_ref[...],
                             preferred_element_type=f32)

    return pl.pallas_call(
        body, grid=(N_PAD // BLK,),
        in_specs=[pl.BlockSpec((BLK, 128), lambda i: (i, 0)),
                  pl.BlockSpec((BLK, 128), lambda i: (i, 0)),
                  pl.BlockSpec((BLK, 16), lambda i: (i, 0)),
                  pl.BlockSpec((BLK, 16), lambda i: (i, 0)),
                  pl.BlockSpec((16, 128), lambda i: (0, 0)),
                  pl.BlockSpec((128, 48), lambda i: (0, 0)),
                  pl.BlockSpec((48, 16), lambda i: (0, 0)),
                  pl.BlockSpec((48, 16), lambda i: (0, 0)),
                  pl.BlockSpec((16, 16), lambda i: (0, 0))],
        out_specs=[pl.BlockSpec((BLK, 48), lambda i: (i, 0)),
                   pl.BlockSpec((BLK, 16), lambda i: (i, 0)),
                   pl.BlockSpec((BLK, 16), lambda i: (i, 0)),
                   pl.BlockSpec((1, 16), lambda i: (0, 0))],
        out_shape=[jax.ShapeDtypeStruct((N_PAD, 48), f32),
                   jax.ShapeDtypeStruct((N_PAD, 16), f32),
                   jax.ShapeDtypeStruct((N_PAD, 16), f32),
                   jax.ShapeDtypeStruct((1, 16), f32)],
        scratch_shapes=[pltpu.VMEM((2, 16), f32)],
    )(m0, m1, d0, d1, R8, W1p, Ma1, Mb1, K1)


def _tc_final(m0, m1, d0, d1, S48, R1):
    BLK = 1000

    def body(m0_ref, m1_ref, d0_ref, d1_ref, s_ref, r_ref, o_ref):
        m = m0_ref[...] + m1_ref[...]
        d = d0_ref[...] + d1_ref[...]
        num = jnp.dot(m, s_ref[...], preferred_element_type=f32)
        db = jnp.dot(d, r_ref[...], preferred_element_type=f32)
        o_ref[...] = num / jnp.maximum(db, 1e-30)

    return pl.pallas_call(
        body, grid=(N_NODES // BLK,),
        in_specs=[pl.BlockSpec((BLK, 48), lambda i: (i, 0)),
                  pl.BlockSpec((BLK, 48), lambda i: (i, 0)),
                  pl.BlockSpec((BLK, 16), lambda i: (i, 0)),
                  pl.BlockSpec((BLK, 16), lambda i: (i, 0)),
                  pl.BlockSpec((48, 40), lambda i: (0, 0)),
                  pl.BlockSpec((16, 40), lambda i: (0, 0))],
        out_specs=pl.BlockSpec((BLK, 40), lambda i: (i, 0)),
        out_shape=jax.ShapeDtypeStruct((N_NODES, 40), f32),
    )(m0, m1, d0, d1, S48, R1)


def _make_sc_edge_pass(ncols, per_head, den_spmem):
    njs = ncols // 16
    mesh = plsc.VectorSubcoreMesh(core_axis_name="c", subcore_axis_name="s",
                                  num_cores=NC, num_subcores=NS)
    if den_spmem:
        den_out = jax.ShapeDtypeStruct((NC, N_PAD, 16), f32)
        den_scr = pltpu.VMEM_SHARED((N_PAD, 16), f32)
    else:
        den_out = jax.ShapeDtypeStruct((NC, NS, N_PAD * 8), f32)
        den_scr = pltpu.VMEM((N_PAD * 8,), f32)
    out_type = (den_out, jax.ShapeDtypeStruct((NC, N_PAD, ncols), f32))
    eebuf_scr = (pltpu.VMEM((2, B, 16), f32) if den_spmem
                 else pltpu.VMEM((1, 1, 16), f32))
    scratch = [
        pltpu.VMEM((4, B), jnp.int32),
        pltpu.VMEM((4, B), jnp.int32),
        pltpu.VMEM((2, B, 16), f32),
        pltpu.VMEM((2, B, 16), f32),
        eebuf_scr,
        pltpu.VMEM((2, B, ncols), f32),
        pltpu.VMEM((16,), f32),
        den_scr,
        pltpu.VMEM_SHARED((N_PAD, ncols), f32),
    ] + [pltpu.SemaphoreType.DMA] * 12

    @functools.partial(pl.kernel, mesh=mesh, out_type=out_type,
                       scratch_types=scratch,
                       compiler_params=pltpu.CompilerParams(
                           use_tc_tiling_on_sc=False,
                           needs_layout_passes=False))
    def sc_pass(h_hbm, ta_hbm, tb_hbm, cvec_hbm, src_hbm, dst_hbm,
                den_hbm, msg_hbm,
                sidx, didx, r1, r2, eebuf, hbuf, cbuf, den, msg_sp,
                semt0, semt1, semh0, semh1,
                semm0, semm1, semd0, semd1,
                semi0, semi1, semi2, semi3):
        semt = (semt0, semt1)
        semh = (semh0, semh1)
        semm = (semm0, semm1)
        semd = (semd0, semd1)
        semi = (semi0, semi1, semi2, semi3)
        cid = lax.axis_index("c")
        sid = lax.axis_index("s")
        w = sid * NC + cid
        zv = jnp.zeros((16,), f32)
        lanes = jnp.arange(16, dtype=jnp.int32)
        m8 = lanes < 8

        def zrow(rn, carry):
            for j in range(njs):
                hbuf[0, rn, pl.ds(j * 16, 16)] = zv
            if den_spmem:
                eebuf[0, rn, :] = zv
            return carry

        lax.fori_loop(0, B, zrow, 0)
        zbase = sid * ZR
        if den_spmem:
            for t in range(ZR // B):
                pltpu.sync_copy(eebuf.at[0],
                                den.at[pl.ds(zbase + t * B, B), :])
        else:
            def zden(i, carry):
                den[pl.ds(16 * i, 16)] = zv
                return carry

            lax.fori_loop(0, N_PAD * 8 // 16, zden, 0)
        for t in range(ZR // B):
            pltpu.sync_copy(hbuf.at[0], msg_sp.at[pl.ds(zbase + t * B, B), :])
        plsc.subcore_barrier()
        pltpu.sync_copy(cvec_hbm.at[0], cbuf)
        cv = cbuf[...]

        def fire_idx(k2, q):
            base = w * EW + k2 * B
            pltpu.async_copy(src_hbm.at[pl.ds(base, B)], sidx.at[q], semi[q])
            pltpu.async_copy(dst_hbm.at[pl.ds(base, B)], didx.at[q], semi[q])

        def wait_idx(q):
            pltpu.make_async_copy(src_hbm.at[pl.ds(0, B)], sidx.at[q],
                                  semi[q]).wait()
            pltpu.make_async_copy(dst_hbm.at[pl.ds(0, B)], didx.at[q],
                                  semi[q]).wait()

        def fire_gather(q, e, s):
            pltpu.async_copy(ta_hbm.at[sidx.at[q]], r1.at[e], semt[e])
            pltpu.async_copy(tb_hbm.at[didx.at[q]], r2.at[e], semt[e])
            pltpu.async_copy(h_hbm.at[sidx.at[q]], hbuf.at[s], semh[s])

        def wait_gather(e, s):
            pltpu.make_async_copy(ta_hbm.at[sidx.at[0]], r1.at[e],
                                  semt[e]).wait()
            pltpu.make_async_copy(tb_hbm.at[didx.at[0]], r2.at[e],
                                  semt[e]).wait()
            pltpu.make_async_copy(h_hbm.at[sidx.at[0]], hbuf.at[s],
                                  semh[s]).wait()

        def drain_msg(s):
            pltpu.make_async_copy(hbuf.at[s], msg_sp.at[didx.at[0]],
                                  semm[s]).wait()

        def drain_den(e):
            pltpu.make_async_copy(eebuf.at[e], den.at[didx.at[0]],
                                  semd[e]).wait()

        def edge(e, dv, j, rn):
            v = r1[e, rn, :] + r2[e, rn, :]
            v = jnp.where(v > 0.0, v, 0.2 * v)
            ee = jnp.exp(v - cv)
            if den_spmem:
                eebuf[e, rn, :] = ee
            else:
                r1[e, rn, :] = ee
                idxv = dv[j] * 8 + lanes
                plsc.addupdate_scatter(den, [idxv], ee, mask=m8)

        def process(k, km, dm, dd, fi, nx):
            s = km & 1
            s1 = (km + 1) & 1
            e = km & 1
            e1 = (km + 1) & 1
            qi = km & 3
            qi1 = (km + 1) & 3
            qi2 = (km + 2) & 3
            if dm:
                drain_msg(s1)
            if fi:
                fire_idx(k + 2, qi2)
            if nx:
                wait_idx(qi1)
                fire_gather(qi1, e1, s1)
            wait_gather(e, s)
            if dd and den_spmem:
                drain_den(e1)

            def egrp(g, c2):
                dv = didx[qi, pl.ds(g * 16, 16)]
                for j in range(16):
                    edge(e, dv, j, g * 16 + j)
                return c2

            lax.fori_loop(0, B // 16, egrp, 0)
            if B % 16:
                dvt = didx[qi, pl.ds(B - 16, 16)]
                for j in range(16 - (B % 16), 16):
                    edge(e, dvt, j, B - 16 + j)
            if den_spmem:
                pltpu.async_copy(eebuf.at[e], den.at[didx.at[qi]], semd[e],
                                 add=True)

            eref = eebuf if den_spmem else r1

            def mrow(rn, c3):
                ev = eref[e, rn, :]
                for j in range(njs):
                    sc = ev[j if per_head else 0]
                    hbuf[s, rn, pl.ds(j * 16, 16)] = (
                        hbuf[s, rn, pl.ds(j * 16, 16)] * sc)
                return c3

            lax.fori_loop(0, B, mrow, 0, unroll=2)
            pltpu.async_copy(hbuf.at[s], msg_sp.at[didx.at[qi]], semm[s],
                             add=True)

        fire_idx(0, 0)
        fire_idx(1, 1)
        wait_idx(0)
        fire_gather(0, 0, 0)
        process(0, 0, False, False, True, True)
        process(1, 1, True, True, True, True)
        process(2, 2, True, True, True, True)
        process(3, 3, True, True, True, True)
        NSTEADY = (NCHUNK - 4 - 3) // 4

        def body4(g, carry):
            kb = 4 + 4 * g
            for j in range(4):
                process(kb + j, 4 + j, True, True, True, True)
            return carry

        lax.fori_loop(0, NSTEADY, body4, 0)
        for kp in range(4 + 4 * NSTEADY, NCHUNK):
            process(kp, kp, True, True,
                    kp + 2 <= NCHUNK - 1, kp + 1 <= NCHUNK - 1)
        drain_msg((NCHUNK - 1) & 1)
        if den_spmem:
            drain_den((NCHUNK - 1) & 1)
            pltpu.sync_copy(den.at[pl.ds(zbase, ZR), :],
                            den_hbm.at[cid, pl.ds(zbase, ZR), :])
        else:
            pltpu.sync_copy(den, den_hbm.at[cid, sid])
        plsc.subcore_barrier()
        pltpu.sync_copy(msg_sp.at[pl.ds(zbase, ZR), :],
                        msg_hbm.at[cid, pl.ds(zbase, ZR), :])

    return sc_pass


_sc_pass_l0 = _make_sc_edge_pass(128, True, True)
_sc_pass_l1 = _make_sc_edge_pass(48, False, True)


def kernel(features, W0, attn_l0, attn_r0, W1, attn_l1, attn_r1, edge_index):
    xpad = jnp.zeros((N_PAD, 128), f32).at[:N_NODES].set(features)
    eye8 = jnp.eye(8, dtype=f32)
    z8 = jnp.zeros((128, 8), f32)
    Mta = jnp.concatenate(
        [(attn_l0[:, :, None] * eye8[:, None, :]).reshape(128, 8), z8], axis=1)
    Mtb = jnp.concatenate(
        [(attn_r0[:, :, None] * eye8[:, None, :]).reshape(128, 8), z8], axis=1)
    K16 = jnp.tile(eye8, (2, 2))
    R8 = jnp.concatenate([jnp.kron(eye8, jnp.ones((1, 16), f32)),
                          jnp.zeros((8, 128), f32)], axis=0)
    W1p = jnp.concatenate([W1, jnp.zeros((128, 8), f32)], axis=1)
    Ma1 = jnp.zeros((48, 16), f32).at[:40, 0].set(attn_l1[0])
    Mb1 = jnp.zeros((48, 16), f32).at[:40, 0].set(attn_r1[0])
    K1 = jnp.zeros((16, 16), f32).at[0, :].set(1.0)
    S48 = jnp.eye(48, dtype=f32)[:, :40]
    R1 = jnp.zeros((16, 40), f32).at[0, :].set(1.0)

    src = edge_index[0]
    dst = edge_index[1]
    h, ta, tb, cvec = _tc_prologue(xpad, W0, Mta, Mtb, K16)
    den, msg = _sc_pass_l0(h, ta, tb, cvec, src, dst)
    h1p, ta1, tb1, c1vec = _tc_mid(msg[0], msg[1], den[0], den[1],
                                   R8, W1p, Ma1, Mb1, K1)
    den1, msg1 = _sc_pass_l1(h1p, ta1, tb1, c1vec, src, dst)
    out = _tc_final(msg1[0, :N_NODES], msg1[1, :N_NODES],
                    den1[0, :N_NODES], den1[1, :N_NODES], S48, R1)
    return out
